# baseline (speedup 1.0000x reference)
import functools

import jax
import jax.numpy as jnp
from jax import lax
from jax.experimental import pallas as pl
from jax.experimental.pallas import tpu as pltpu
from jax.experimental.pallas import tpu_sc as plsc


def _make_sc_broadcast(batch, seq_len, dim, dtype):
    info = plsc.get_sparse_core_info()
    nw = info.num_cores * info.num_subcores
    rows_per_w = seq_len // nw
    chunk = 64
    n_chunks = rows_per_w // chunk

    mesh = plsc.VectorSubcoreMesh(core_axis_name="c", subcore_axis_name="s")

    @functools.partial(
        pl.kernel,
        mesh=mesh,
        out_type=jax.ShapeDtypeStruct((batch, seq_len, dim), dtype),
        scratch_types=[
            pltpu.VMEM((chunk, dim), dtype),
            pltpu.SemaphoreType.DMA,
            pltpu.SemaphoreType.DMA,
        ],
    )
    def k(table_hbm, out_hbm, buf, sem, semd):
        wid = lax.axis_index("s") * info.num_cores + lax.axis_index("c")
        base = wid * rows_per_w

        def body(i, _):
            r0 = base + i * chunk
            direct = [
                pltpu.async_copy(table_hbm.at[pl.ds(r0, chunk)],
                                 out_hbm.at[b, pl.ds(r0, chunk)], semd)
                for b in (2, 3)
            ]
            pltpu.sync_copy(table_hbm.at[pl.ds(r0, chunk)], buf)
            staged = [
                pltpu.async_copy(buf, out_hbm.at[b, pl.ds(r0, chunk)], sem)
                for b in (0, 1)
            ]
            for c in staged + direct:
                c.wait()
            return _

        lax.fori_loop(0, n_chunks, body, None)

    return k


def kernel(x, symbol_library):
    batch, seq_len, dim = x.shape
    k = _make_sc_broadcast(batch, seq_len, dim, symbol_library.dtype)
    return k(symbol_library)


# speedup vs baseline: 27.9103x; 27.9103x over previous
import functools

import jax
import jax.numpy as jnp
from jax import lax
from jax.experimental import pallas as pl
from jax.experimental.pallas import tpu as pltpu
from jax.experimental.pallas import tpu_sc as plsc


def _make_sc_broadcast(batch, seq_len, dim, dtype):
    info = plsc.get_sparse_core_info()
    nw = info.num_cores * info.num_subcores
    rows_per_w = seq_len // nw
    chunk = 64
    n_chunks = rows_per_w // chunk

    mesh = plsc.VectorSubcoreMesh(core_axis_name="c", subcore_axis_name="s")

    @functools.partial(
        pl.kernel,
        mesh=mesh,
        out_type=jax.ShapeDtypeStruct((batch, seq_len, dim), dtype),
        scratch_types=[
            pltpu.VMEM((chunk, dim), dtype),
            pltpu.SemaphoreType.DMA,
        ],
    )
    def k(table_hbm, out_hbm, buf, sem):
        wid = lax.axis_index("s") * info.num_cores + lax.axis_index("c")
        base = wid * rows_per_w

        def body(i, _):
            r0 = base + i * chunk
            pltpu.sync_copy(table_hbm.at[pl.ds(r0, chunk)], buf)
            copies = [
                pltpu.async_copy(buf, out_hbm.at[b, pl.ds(r0, chunk)], sem)
                for b in range(batch)
            ]
            for c in copies:
                c.wait()
            return _

        lax.fori_loop(0, n_chunks, body, None)

    return k


def kernel(x, symbol_library):
    batch, seq_len, _ = x.shape
    max_len, dim = symbol_library.shape
    k = _make_sc_broadcast(batch, seq_len, dim, symbol_library.dtype)
    return k(symbol_library)
